# Initial kernel scaffold; baseline (speedup 1.0000x reference)
import jax
import jax.numpy as jnp
from jax.experimental import pallas as pl


def kernel(x, edge_index, W0, wih0, whh0, bih0, bhh0, W1, wih1, whh1, bih1, bhh1):
    raise NotImplementedError("write your pallas kernel here")


# speedup vs baseline: 1.4892x; 1.4892x over previous
import functools

import jax
import jax.numpy as jnp
from jax import lax
from jax.experimental import pallas as pl
from jax.experimental.pallas import tpu as pltpu
from jax.experimental.pallas import tpu_sc as plsc

N = 10000
C = 128
GG_STEPS = 4
NC = 2
NS = 16
NW = NC * NS
K = 128

R = 320
N_PAD = NW * R

HALF_E = 160000
W_A = 10080
W_HI = 110880
W_B = 9840
W_END = 150240


def _seg_sum_body(m_hbm, srcs_hbm, dsts_hbm, starts_hbm, zeros_hbm, out_hbm,
                  starts_v, sidx_v, didx_v, rows_v, acc_v, work_v, sem):
    cid = lax.axis_index("c")
    sid = lax.axis_index("s")
    wid = cid * NS + sid
    pltpu.sync_copy(zeros_hbm, acc_v)
    pltpu.sync_copy(zeros_hbm, work_v)
    pltpu.sync_copy(starts_hbm, starts_v.at[pl.ds(0, 40)])
    s_w = starts_v[pl.ds(wid, 16)][0]
    e_w = starts_v[pl.ds(wid + 1, 16)][0]
    c0 = s_w // K
    c1 = (e_w + K - 1) // K
    base_row = wid * R

    def _merge():
        @pl.loop(0, R + 1)
        def _row(rr):
            for j in range(C // 16):
                sl = pl.ds(j * 16, 16)
                acc_v[rr, sl] = acc_v[rr, sl] + work_v[rr, sl]
                work_v[rr, sl] = jnp.zeros((16,), jnp.float32)

    @pl.loop(c0, c1)
    def _chunk(c):
        off = c * K
        pltpu.sync_copy(srcs_hbm.at[pl.ds(off, K)], sidx_v)
        pltpu.sync_copy(dsts_hbm.at[pl.ds(off, K)], didx_v.at[pl.ds(0, K)])
        pltpu.async_copy(m_hbm.at[sidx_v], rows_v, sem).wait()

        @pl.loop(0, K)
        def _edge(e):
            g = off + e
            g2 = g - jnp.where(g >= HALF_E, HALF_E, 0)
            is_b = (g > 0) & (
                (g2 == 0)
                | ((g2 <= W_HI) & (g2 % W_A == 0))
                | ((g2 > W_HI) & (g2 <= W_END) & ((g2 - W_HI) % W_B == 0)))

            @pl.when(is_b)
            def _():
                _merge()

            ld = didx_v[pl.ds(e, 16)][0] - base_row
            own = (ld >= 0) & (ld < R)
            ld = jnp.where(own, ld, R)
            for j in range(C // 16):
                sl = pl.ds(j * 16, 16)
                work_v[ld, sl] = work_v[ld, sl] + rows_v[e, sl]

    _merge()
    pltpu.sync_copy(acc_v.at[pl.ds(0, R)],
                    out_hbm.at[pl.ds(base_row, R)])


def _make_seg_sum():
    mesh = plsc.VectorSubcoreMesh(core_axis_name="c", subcore_axis_name="s")
    return pl.kernel(
        _seg_sum_body,
        out_type=jax.ShapeDtypeStruct((N_PAD, C), jnp.float32),
        mesh=mesh,
        scratch_types=[
            pltpu.VMEM((56,), jnp.int32),
            pltpu.VMEM((K,), jnp.int32),
            pltpu.VMEM((K + 16,), jnp.int32),
            pltpu.VMEM((K, C), jnp.float32),
            pltpu.VMEM((R + 1, C), jnp.float32),
            pltpu.VMEM((R + 1, C), jnp.float32),
            pltpu.SemaphoreType.DMA,
        ],
    )


_BLK = 1000


def _mm_body(x_ref, w_ref, o_ref):
    o_ref[...] = jnp.dot(x_ref[...], w_ref[...],
                         preferred_element_type=jnp.float32)


def _matmul(x, w):
    return pl.pallas_call(
        _mm_body,
        grid=(N // _BLK,),
        in_specs=[
            pl.BlockSpec((_BLK, C), lambda i: (i, 0)),
            pl.BlockSpec((C, C), lambda i: (0, 0)),
        ],
        out_specs=pl.BlockSpec((_BLK, C), lambda i: (i, 0)),
        out_shape=jax.ShapeDtypeStruct((N, C), jnp.float32),
        compiler_params=pltpu.CompilerParams(
            dimension_semantics=("parallel",)),
    )(x, w)


def _step_body(relu, has_m, agg_ref, h_ref, wihT_ref, whhT_ref,
               bih_ref, bhh_ref, *rest):
    if has_m:
        wn_ref, h_out, m_out = rest
    else:
        (h_out,) = rest
    agg = agg_ref[...]
    h = h_ref[...]
    gi = jnp.dot(agg, wihT_ref[...], preferred_element_type=jnp.float32)
    gi = gi + bih_ref[...]
    gh = jnp.dot(h, whhT_ref[...], preferred_element_type=jnp.float32)
    gh = gh + bhh_ref[...]
    r = jax.nn.sigmoid(gi[:, :C] + gh[:, :C])
    z = jax.nn.sigmoid(gi[:, C:2 * C] + gh[:, C:2 * C])
    n = jnp.tanh(gi[:, 2 * C:] + r * gh[:, 2 * C:])
    hn = (1.0 - z) * n + z * h
    if relu:
        hn = jnp.maximum(hn, 0.0)
    h_out[...] = hn
    if has_m:
        m_out[...] = jnp.dot(hn, wn_ref[...],
                             preferred_element_type=jnp.float32)


def _gru_step(agg, h, wihT, whhT, bih, bhh, wn, relu):
    has_m = wn is not None
    in_specs = [
        pl.BlockSpec((_BLK, C), lambda i: (i, 0)),
        pl.BlockSpec((_BLK, C), lambda i: (i, 0)),
        pl.BlockSpec((C, 3 * C), lambda i: (0, 0)),
        pl.BlockSpec((C, 3 * C), lambda i: (0, 0)),
        pl.BlockSpec((1, 3 * C), lambda i: (0, 0)),
        pl.BlockSpec((1, 3 * C), lambda i: (0, 0)),
    ]
    args = [agg, h, wihT, whhT, bih, bhh]
    if has_m:
        in_specs.append(pl.BlockSpec((C, C), lambda i: (0, 0)))
        args.append(wn)
        out_specs = (pl.BlockSpec((_BLK, C), lambda i: (i, 0)),) * 2
        out_shape = (jax.ShapeDtypeStruct((N, C), jnp.float32),) * 2
    else:
        out_specs = pl.BlockSpec((_BLK, C), lambda i: (i, 0))
        out_shape = jax.ShapeDtypeStruct((N, C), jnp.float32)
    return pl.pallas_call(
        functools.partial(_step_body, relu, has_m),
        grid=(N // _BLK,),
        in_specs=in_specs,
        out_specs=out_specs,
        out_shape=out_shape,
        compiler_params=pltpu.CompilerParams(
            dimension_semantics=("parallel",)),
    )(*args)


def kernel(x, edge_index, W0, wih0, whh0, bih0, bhh0,
           W1, wih1, whh1, bih1, bhh1):
    src = edge_index[0].astype(jnp.int32)
    dst = edge_index[1].astype(jnp.int32)
    order = jnp.argsort(dst, stable=True)
    src_s = src[order]
    dst_s = dst[order]
    e_pad = -(-src.shape[0] // K) * K
    if e_pad > src.shape[0]:
        extra = e_pad - src.shape[0]
        src_s = jnp.concatenate([src_s, jnp.zeros((extra,), jnp.int32)])
        dst_s = jnp.concatenate(
            [dst_s, jnp.full((extra,), N_PAD, jnp.int32)])
    starts = jnp.searchsorted(
        dst_s[:src.shape[0]],
        R * jnp.arange(33, dtype=jnp.int32)).astype(jnp.int32)
    starts = jnp.concatenate(
        [starts, jnp.full((7,), src.shape[0], jnp.int32)])
    zeros = jnp.zeros((R + 1, C), jnp.float32)
    seg_sum = _make_seg_sum()

    params = (
        (W0, wih0.T, whh0.T, bih0.reshape(1, -1), bhh0.reshape(1, -1)),
        (W1, wih1.T, whh1.T, bih1.reshape(1, -1), bhh1.reshape(1, -1)),
    )

    h = x
    m = _matmul(h, W0[0])
    for layer in range(2):
        _, wihT, whhT, bih, bhh = params[layer]
        for s in range(GG_STEPS):
            agg = seg_sum(m, src_s, dst_s, starts, zeros)
            last = layer == 1 and s == GG_STEPS - 1
            relu = layer == 0 and s == GG_STEPS - 1
            if last:
                wn = None
            elif s == GG_STEPS - 1:
                wn = params[1][0][0]
            else:
                wn = params[layer][0][s + 1]
            out = _gru_step(agg, h, wihT, whhT, bih, bhh, wn, relu)
            if last:
                h = out
            else:
                h, m = out
    return h


# speedup vs baseline: 1.7982x; 1.2075x over previous
import functools

import jax
import jax.numpy as jnp
from jax import lax
from jax.experimental import pallas as pl
from jax.experimental.pallas import tpu as pltpu
from jax.experimental.pallas import tpu_sc as plsc

N = 10000
C = 128
GG_STEPS = 4
NC = 2
NS = 16
NW = NC * NS
K = 128

R = 320
N_PAD = NW * R

HALF_E = 160000
W_A = 10080
W_HI = 110880
W_B = 9840
W_END = 150240
_BOUNDS = ([W_A * k for k in range(1, 12)]
           + [W_HI + W_B * j for j in range(1, 5)])
_BOUNDS = _BOUNDS + [HALF_E] + [HALF_E + b for b in _BOUNDS]
_BCHUNKS = sorted({b // K for b in _BOUNDS})


def _seg_sum_body(m_hbm, srcs_hbm, dsts_hbm, starts_hbm, zeros_hbm, out_hbm,
                  starts_v, sidx_v, didx_v, rows_v, acc_v, work_v, sem):
    cid = lax.axis_index("c")
    sid = lax.axis_index("s")
    wid = cid * NS + sid
    pltpu.sync_copy(zeros_hbm, acc_v)
    pltpu.sync_copy(zeros_hbm, work_v)
    pltpu.sync_copy(starts_hbm, starts_v.at[pl.ds(0, 40)])
    s_w = starts_v[pl.ds(wid, 16)][0]
    e_w = starts_v[pl.ds(wid + 1, 16)][0]
    c0 = s_w // K
    c1 = (e_w + K - 1) // K
    base_row = wid * R

    def _merge():
        @pl.loop(0, R + 1)
        def _row(rr):
            for j in range(C // 16):
                sl = pl.ds(j * 16, 16)
                acc_v[rr, sl] = acc_v[rr, sl] + work_v[rr, sl]
                work_v[rr, sl] = jnp.zeros((16,), jnp.float32)

    def _fold_edge(off, e, check):
        if check:
            g = off + e
            g2 = g - jnp.where(g >= HALF_E, HALF_E, 0)
            is_b = (g > 0) & (
                (g2 == 0)
                | ((g2 <= W_HI) & (g2 % W_A == 0))
                | ((g2 > W_HI) & (g2 <= W_END) & ((g2 - W_HI) % W_B == 0)))

            @pl.when(is_b)
            def _():
                _merge()

        ld = didx_v[pl.ds(e, 16)][0] - base_row
        own = (ld >= 0) & (ld < R)
        ld = jnp.where(own, ld, R)
        for j in range(C // 16):
            sl = pl.ds(j * 16, 16)
            work_v[ld, sl] = work_v[ld, sl] + rows_v[e, sl]

    @pl.loop(c0, c1)
    def _chunk(c):
        off = c * K
        pltpu.sync_copy(srcs_hbm.at[pl.ds(off, K)], sidx_v)
        pltpu.sync_copy(dsts_hbm.at[pl.ds(off, K)], didx_v.at[pl.ds(0, K)])
        pltpu.async_copy(m_hbm.at[sidx_v], rows_v, sem).wait()

        has_b = functools.reduce(
            lax.bitwise_or, [c == bc for bc in _BCHUNKS])

        @pl.when(has_b)
        def _slow():
            @pl.loop(0, K)
            def _edge(e):
                _fold_edge(off, e, True)

        @pl.when(jnp.logical_not(has_b))
        def _fast():
            @pl.loop(0, K, unroll=4)
            def _edge(e):
                _fold_edge(off, e, False)

    _merge()
    pltpu.sync_copy(acc_v.at[pl.ds(0, R)],
                    out_hbm.at[pl.ds(base_row, R)])


def _make_seg_sum():
    mesh = plsc.VectorSubcoreMesh(core_axis_name="c", subcore_axis_name="s")
    return pl.kernel(
        _seg_sum_body,
        out_type=jax.ShapeDtypeStruct((N_PAD, C), jnp.float32),
        mesh=mesh,
        scratch_types=[
            pltpu.VMEM((56,), jnp.int32),
            pltpu.VMEM((K,), jnp.int32),
            pltpu.VMEM((K + 16,), jnp.int32),
            pltpu.VMEM((K, C), jnp.float32),
            pltpu.VMEM((R + 1, C), jnp.float32),
            pltpu.VMEM((R + 1, C), jnp.float32),
            pltpu.SemaphoreType.DMA,
        ],
    )


_BLK = 1000


def _mm_body(x_ref, w_ref, o_ref):
    o_ref[...] = jnp.dot(x_ref[...], w_ref[...],
                         preferred_element_type=jnp.float32)


def _matmul(x, w):
    return pl.pallas_call(
        _mm_body,
        grid=(N // _BLK,),
        in_specs=[
            pl.BlockSpec((_BLK, C), lambda i: (i, 0)),
            pl.BlockSpec((C, C), lambda i: (0, 0)),
        ],
        out_specs=pl.BlockSpec((_BLK, C), lambda i: (i, 0)),
        out_shape=jax.ShapeDtypeStruct((N, C), jnp.float32),
        compiler_params=pltpu.CompilerParams(
            dimension_semantics=("parallel",)),
    )(x, w)


def _step_body(relu, has_m, agg_ref, h_ref, wihT_ref, whhT_ref,
               bih_ref, bhh_ref, *rest):
    if has_m:
        wn_ref, h_out, m_out = rest
    else:
        (h_out,) = rest
    agg = agg_ref[...]
    h = h_ref[...]
    gi = jnp.dot(agg, wihT_ref[...], preferred_element_type=jnp.float32)
    gi = gi + bih_ref[...]
    gh = jnp.dot(h, whhT_ref[...], preferred_element_type=jnp.float32)
    gh = gh + bhh_ref[...]
    r = jax.nn.sigmoid(gi[:, :C] + gh[:, :C])
    z = jax.nn.sigmoid(gi[:, C:2 * C] + gh[:, C:2 * C])
    n = jnp.tanh(gi[:, 2 * C:] + r * gh[:, 2 * C:])
    hn = (1.0 - z) * n + z * h
    if relu:
        hn = jnp.maximum(hn, 0.0)
    h_out[...] = hn
    if has_m:
        m_out[...] = jnp.dot(hn, wn_ref[...],
                             preferred_element_type=jnp.float32)


def _gru_step(agg, h, wihT, whhT, bih, bhh, wn, relu):
    has_m = wn is not None
    in_specs = [
        pl.BlockSpec((_BLK, C), lambda i: (i, 0)),
        pl.BlockSpec((_BLK, C), lambda i: (i, 0)),
        pl.BlockSpec((C, 3 * C), lambda i: (0, 0)),
        pl.BlockSpec((C, 3 * C), lambda i: (0, 0)),
        pl.BlockSpec((1, 3 * C), lambda i: (0, 0)),
        pl.BlockSpec((1, 3 * C), lambda i: (0, 0)),
    ]
    args = [agg, h, wihT, whhT, bih, bhh]
    if has_m:
        in_specs.append(pl.BlockSpec((C, C), lambda i: (0, 0)))
        args.append(wn)
        out_specs = (pl.BlockSpec((_BLK, C), lambda i: (i, 0)),) * 2
        out_shape = (jax.ShapeDtypeStruct((N, C), jnp.float32),) * 2
    else:
        out_specs = pl.BlockSpec((_BLK, C), lambda i: (i, 0))
        out_shape = jax.ShapeDtypeStruct((N, C), jnp.float32)
    return pl.pallas_call(
        functools.partial(_step_body, relu, has_m),
        grid=(N // _BLK,),
        in_specs=in_specs,
        out_specs=out_specs,
        out_shape=out_shape,
        compiler_params=pltpu.CompilerParams(
            dimension_semantics=("parallel",)),
    )(*args)


def kernel(x, edge_index, W0, wih0, whh0, bih0, bhh0,
           W1, wih1, whh1, bih1, bhh1):
    src = edge_index[0].astype(jnp.int32)
    dst = edge_index[1].astype(jnp.int32)
    order = jnp.argsort(dst, stable=True)
    src_s = src[order]
    dst_s = dst[order]
    e_pad = -(-src.shape[0] // K) * K
    if e_pad > src.shape[0]:
        extra = e_pad - src.shape[0]
        src_s = jnp.concatenate([src_s, jnp.zeros((extra,), jnp.int32)])
        dst_s = jnp.concatenate(
            [dst_s, jnp.full((extra,), N_PAD, jnp.int32)])
    starts = jnp.searchsorted(
        dst_s[:src.shape[0]],
        R * jnp.arange(33, dtype=jnp.int32)).astype(jnp.int32)
    starts = jnp.concatenate(
        [starts, jnp.full((7,), src.shape[0], jnp.int32)])
    zeros = jnp.zeros((R + 1, C), jnp.float32)
    seg_sum = _make_seg_sum()

    params = (
        (W0, wih0.T, whh0.T, bih0.reshape(1, -1), bhh0.reshape(1, -1)),
        (W1, wih1.T, whh1.T, bih1.reshape(1, -1), bhh1.reshape(1, -1)),
    )

    h = x
    m = _matmul(h, W0[0])
    for layer in range(2):
        _, wihT, whhT, bih, bhh = params[layer]
        for s in range(GG_STEPS):
            agg = seg_sum(m, src_s, dst_s, starts, zeros)
            last = layer == 1 and s == GG_STEPS - 1
            relu = layer == 0 and s == GG_STEPS - 1
            if last:
                wn = None
            elif s == GG_STEPS - 1:
                wn = params[1][0][0]
            else:
                wn = params[layer][0][s + 1]
            out = _gru_step(agg, h, wihT, whhT, bih, bhh, wn, relu)
            if last:
                h = out
            else:
                h, m = out
    return h
